# baseline (speedup 1.0000x reference)
import functools

import jax
import jax.numpy as jnp
from jax import lax
from jax.experimental import pallas as pl
from jax.experimental.pallas import tpu as pltpu
from jax.experimental.pallas import tpu_sc as plsc

N = 10000
NPAD = 10240
F = 128
E = 320000
EPAD = 327680
ER = EPAD // 128
NT = 32
TROWS = ER // NT
CHROWS = 4
NCH = TROWS // CHROWS
NSLICE = NPAD // 16
NH = NPAD // 2
NHT = NH + 128
RSLICE = NHT // 16
TROWS2 = ER // 16
NCH2 = TROWS2 // CHROWS


def _sc_deg_body(m_hbm, srcp, dstp, z1_hbm, degp_hbm, dste_hbm,
                 m_v, src_c, dst_c, d0_c, d1_c, val_c, sdeg):
    c = lax.axis_index("c")
    s = lax.axis_index("s")
    wid = c * 16 + s
    pltpu.sync_copy(m_hbm, m_v)
    pltpu.sync_copy(z1_hbm.at[pl.ds(s * NSLICE, NSLICE)],
                    sdeg.at[pl.ds(s * NSLICE, NSLICE)])
    plsc.subcore_barrier()

    def chunk(j, carry):
        rb = wid * TROWS + j * CHROWS
        pltpu.sync_copy(srcp.at[pl.ds(rb, CHROWS)], src_c)
        pltpu.sync_copy(dstp.at[pl.ds(rb, CHROWS)], dst_c)
        for g in range(CHROWS):
            for i in range(8):
                sl = pl.ds(i * 16, 16)
                s16 = src_c[g, sl]
                d16 = dst_c[g, sl]
                ms = plsc.load_gather(m_v, [s16])
                md = plsc.load_gather(m_v, [d16])
                vld = ms * md
                dump = NH + (d16 & 127)
                ok = vld > 0
                d0_c[g, sl] = jnp.where(ok & (d16 < NH), d16, dump)
                d1_c[g, sl] = jnp.where(ok & (d16 >= NH), d16 - NH, dump)
                val_c[g, sl] = vld.astype(jnp.float32)
        pltpu.sync_copy(d0_c, dste_hbm.at[0, pl.ds(rb, CHROWS)])
        pltpu.sync_copy(d1_c, dste_hbm.at[1, pl.ds(rb, CHROWS)])
        for g in range(CHROWS):
            pltpu.sync_copy(val_c.at[g], sdeg.at[dst_c.at[g]], add=True)
        return carry

    lax.fori_loop(0, NCH, chunk, 0)
    plsc.subcore_barrier()
    pltpu.sync_copy(sdeg.at[pl.ds(s * NSLICE, NSLICE)],
                    degp_hbm.at[c, pl.ds(s * NSLICE, NSLICE)])


@functools.lru_cache(maxsize=None)
def _get_sc_deg():
    mesh = plsc.VectorSubcoreMesh(core_axis_name="c", subcore_axis_name="s")
    return pl.kernel(
        _sc_deg_body,
        out_type=(
            jax.ShapeDtypeStruct((2, NPAD), jnp.float32),
            jax.ShapeDtypeStruct((2, ER, 128), jnp.int32),
        ),
        mesh=mesh,
        compiler_params=pltpu.CompilerParams(needs_layout_passes=False),
        scratch_types=[
            pltpu.VMEM((NPAD,), jnp.int32),
            pltpu.VMEM((CHROWS, 128), jnp.int32),
            pltpu.VMEM((CHROWS, 128), jnp.int32),
            pltpu.VMEM((CHROWS, 128), jnp.int32),
            pltpu.VMEM((CHROWS, 128), jnp.int32),
            pltpu.VMEM((CHROWS, 128), jnp.float32),
            pltpu.VMEM_SHARED((NPAD,), jnp.float32),
        ],
    )


SUP = 16
NSUPT = ER // SUP


def _sc_feat_body(tbl_hbm, srcp, dste, meta_hbm, z2_hbm, acc_hbm,
                  meta_s, src_c, dst_c, rows_a, rows_b, acc, gsem, ssem):
    c = lax.axis_index("c")
    s = lax.axis_index("s")
    pltpu.sync_copy(meta_hbm, meta_s)
    pltpu.sync_copy(z2_hbm.at[pl.ds(s * RSLICE, RSLICE)],
                    acc.at[pl.ds(s * RSLICE, RSLICE)])
    plsc.subcore_barrier()
    bufs = (rows_a, rows_b)
    t0f = jnp.max(meta_s[0, :])
    t0c = jnp.max(meta_s[1, :])
    start = jnp.where(c == 0, 0, t0f)
    end = jnp.where(c == 0, t0c, NSUPT)
    n = (end - start - s + 15) // 16

    def sup(j, carry):
        rb = (start + s + j * 16) * SUP
        pltpu.sync_copy(srcp.at[pl.ds(rb, SUP)], src_c)
        pltpu.sync_copy(dste.at[c, pl.ds(rb, SUP)], dst_c)
        pend_sc = [None, None]
        pend_g = [None, None]
        pend_g[0] = pltpu.async_copy(tbl_hbm.at[src_c.at[0]], bufs[0], gsem)
        for g in range(SUP):
            b = g % 2
            if g + 1 < SUP:
                nb = (g + 1) % 2
                if pend_sc[nb] is not None:
                    pend_sc[nb].wait()
                    pend_sc[nb] = None
                pend_g[nb] = pltpu.async_copy(
                    tbl_hbm.at[src_c.at[g + 1]], bufs[nb], gsem)
            pend_g[b].wait()
            pend_sc[b] = pltpu.async_copy(
                bufs[b], acc.at[dst_c.at[g]], ssem, add=True)
        for d in pend_sc:
            if d is not None:
                d.wait()
        return carry

    lax.fori_loop(0, n, sup, 0)
    plsc.subcore_barrier()
    pltpu.sync_copy(acc.at[pl.ds(s * RSLICE, RSLICE)],
                    acc_hbm.at[c, pl.ds(s * RSLICE, RSLICE)])


@functools.lru_cache(maxsize=None)
def _get_sc_feat():
    mesh = plsc.VectorSubcoreMesh(core_axis_name="c", subcore_axis_name="s")
    return pl.kernel(
        _sc_feat_body,
        out_type=jax.ShapeDtypeStruct((2, NHT, F), jnp.float32),
        mesh=mesh,
        compiler_params=pltpu.CompilerParams(needs_layout_passes=False),
        scratch_types=[
            pltpu.VMEM((2, 16), jnp.int32),
            pltpu.VMEM((SUP, 128), jnp.int32),
            pltpu.VMEM((SUP, 128), jnp.int32),
            pltpu.VMEM((128, F), jnp.float32),
            pltpu.VMEM((128, F), jnp.float32),
            pltpu.VMEM_SHARED((NHT, F), jnp.float32),
            pltpu.SemaphoreType.DMA,
            pltpu.SemaphoreType.DMA,
        ],
    )


BR = 512
GB = NPAD // BR
HB = NH // BR


def _dense_a_body(x_ref, w_ref, degp_ref, h_ref, ht_ref, degc_ref):
    deg = degp_ref[0] + degp_ref[1]
    h = jnp.dot(x_ref[...], w_ref[...], preferred_element_type=jnp.float32)
    dinvg = lax.rsqrt(deg + 1.0)
    h_ref[...] = h
    ht_ref[...] = h * dinvg
    degc_ref[...] = deg


_dense_a = pl.pallas_call(
    _dense_a_body,
    grid=(GB,),
    in_specs=[
        pl.BlockSpec((BR, F), lambda i: (i, 0)),
        pl.BlockSpec((F, F), lambda i: (0, 0)),
        pl.BlockSpec((2, BR, 1), lambda i: (0, i, 0)),
    ],
    out_specs=[
        pl.BlockSpec((BR, F), lambda i: (i, 0)),
        pl.BlockSpec((BR, F), lambda i: (i, 0)),
        pl.BlockSpec((BR, 1), lambda i: (i, 0)),
    ],
    out_shape=[
        jax.ShapeDtypeStruct((NPAD, F), jnp.float32),
        jax.ShapeDtypeStruct((NPAD, F), jnp.float32),
        jax.ShapeDtypeStruct((NPAD, 1), jnp.float32),
    ],
)


def _dense_b_body(h_ref, ap_ref, degc_ref, b_ref, H_ref, Hs_ref):
    deg = degc_ref[...]
    dinvg = lax.rsqrt(deg + 1.0)
    A = ap_ref[0]
    H = jnp.maximum(A * dinvg + h_ref[...] * (dinvg * dinvg) + b_ref[...], 0.0)
    H_ref[...] = H
    dinvs = jnp.where(deg > 0, lax.rsqrt(jnp.where(deg > 0, deg, 1.0)), 0.0)
    Hs_ref[...] = H * dinvs


_dense_b = pl.pallas_call(
    _dense_b_body,
    grid=(GB,),
    in_specs=[
        pl.BlockSpec((BR, F), lambda i: (i, 0)),
        pl.BlockSpec((1, BR, F), lambda i: (i // HB, i % HB, 0)),
        pl.BlockSpec((BR, 1), lambda i: (i, 0)),
        pl.BlockSpec((1, F), lambda i: (0, 0)),
    ],
    out_specs=[
        pl.BlockSpec((BR, F), lambda i: (i, 0)),
        pl.BlockSpec((BR, F), lambda i: (i, 0)),
    ],
    out_shape=[
        jax.ShapeDtypeStruct((NPAD, F), jnp.float32),
        jax.ShapeDtypeStruct((NPAD, F), jnp.float32),
    ],
)


def _pool_body(k, H_ref, pp_ref, degc_ref, m_ref, mn_ref, ro_ref):
    deg = degc_ref[...]
    dinvs = jnp.where(deg > 0, lax.rsqrt(jnp.where(deg > 0, deg, 1.0)), 0.0)
    H = H_ref[...]
    P = jnp.concatenate([pp_ref[0, :NH, :], pp_ref[1, :NH, :]], axis=0) * dinvs
    sc = jnp.sum(jnp.abs(H - P), axis=1, keepdims=True)
    key = jnp.where(m_ref[...] > 0, sc, -jnp.inf)
    bi = lax.bitcast_convert_type(key, jnp.int32)
    v = jnp.where(bi >= 0, bi, bi ^ jnp.int32(0x7FFFFFFF))
    w = lax.bitcast_convert_type(v ^ jnp.int32(-2147483648), jnp.uint32)

    def bit_body(b, acc):
        cand = acc | (jnp.uint32(1) << (jnp.uint32(31) - b.astype(jnp.uint32)))
        cnt = jnp.sum((w >= cand).astype(jnp.int32))
        return jnp.where(cnt >= k, cand, acc)

    ustar = lax.fori_loop(0, 32, bit_body, jnp.uint32(0))
    c_gt = jnp.sum((w > ustar).astype(jnp.int32))
    r = k - c_gt
    eq = w == ustar
    idx = lax.broadcasted_iota(jnp.int32, (NPAD, 1), 0)

    def ix_body(b, lohi):
        lo, hi = lohi
        mid = (lo + hi) // 2
        g = jnp.sum((eq & (idx < mid)).astype(jnp.int32))
        take = g >= r
        return (jnp.where(take, lo, mid), jnp.where(take, mid, hi))

    _, jstar = lax.fori_loop(0, 15, ix_body, (jnp.int32(0), jnp.int32(NPAD)))
    mn = (w > ustar) | (eq & (idx < jstar))
    mn_ref[...] = mn.astype(jnp.int32)
    mx = jnp.max(jnp.where(mn, H, -jnp.inf), axis=0, keepdims=True)
    sm = jnp.sum(jnp.where(mn, H, 0.0), axis=0, keepdims=True)
    ro_ref[...] = jnp.concatenate([mx, sm * (1.0 / k)], axis=1)


@functools.lru_cache(maxsize=None)
def _pool_call(k):
    return pl.pallas_call(
        functools.partial(_pool_body, k),
        out_shape=[
            jax.ShapeDtypeStruct((NPAD, 1), jnp.int32),
            jax.ShapeDtypeStruct((1, 2 * F), jnp.float32),
        ],
    )


def _head_body(H_ref, m_ref, x1_ref, x2_ref, w1_ref, v1_ref, w2_ref, v2_ref,
               w3_ref, v3_ref, out_ref):
    mb = m_ref[...] > 0
    H = H_ref[...]
    mx = jnp.max(jnp.where(mb, H, -jnp.inf), axis=0, keepdims=True)
    sm = jnp.sum(jnp.where(mb, H, 0.0), axis=0, keepdims=True)
    x3 = jnp.concatenate([mx, sm * (1.0 / 2500.0)], axis=1)
    z = (jnp.maximum(x1_ref[...], 0.0) + jnp.maximum(x2_ref[...], 0.0)
         + jnp.maximum(x3, 0.0))
    z = jnp.maximum(
        jnp.dot(z, w1_ref[...], preferred_element_type=jnp.float32)
        + v1_ref[...], 0.0)
    z = jnp.maximum(
        jnp.dot(z, w2_ref[...], preferred_element_type=jnp.float32)
        + v2_ref[...], 0.0)
    z = (jnp.dot(z, w3_ref[...], preferred_element_type=jnp.float32)
         + v3_ref[...])
    nrm = jnp.maximum(jnp.sqrt(jnp.sum(z * z)), 1e-12)
    out_ref[...] = z / nrm


_head = pl.pallas_call(
    _head_body,
    out_shape=jax.ShapeDtypeStruct((1, 64), jnp.float32),
)


def _stage(X, W, bvec, m_flat, srcp, dstp, meta, z1, z2, k):
    degp, dste = _get_sc_deg()(m_flat, srcp, dstp, z1)
    h, ht, degc = _dense_a(X, W, degp.reshape(2, NPAD, 1))
    Ap = _get_sc_feat()(ht, srcp, dste, meta, z2)
    H, Hs = _dense_b(h, Ap, degc, bvec.reshape(1, F))
    if k is None:
        return H, None, None
    Pp = _get_sc_feat()(Hs, srcp, dste, meta, z2)
    mn, ro = _pool_call(k)(H, Pp, degc, m_flat.reshape(NPAD, 1))
    return H, mn.reshape(NPAD), ro


def kernel(x, edge_index, batch, W1, b1, W2, b2, W3, b3,
           L1W, L1b, L2W, L2b, L3W, L3b):
    f32 = jnp.float32
    i32 = jnp.int32
    xp = jnp.zeros((NPAD, F), f32).at[:N].set(x)
    src = edge_index[0].astype(i32)
    dst = edge_index[1].astype(i32)
    in0 = dst < NH
    f0 = in0.astype(i32)
    f2 = f0.reshape(E // 128, 128)
    ws = jnp.sum(f2, axis=1)
    cw = jnp.cumsum(ws)
    lane = jnp.cumsum(f2, axis=1)
    cs0 = (lane + (cw - ws)[:, None]).reshape(E)
    c0 = cw[E // 128 - 1]
    ii = jnp.arange(E, dtype=i32)
    pos = jnp.where(in0, cs0 - 1, c0 + ii - cs0)
    packed = jnp.full((EPAD,), (NPAD - 1) << 14, i32).at[pos].set(
        (dst << 14) | src)
    srcp = (packed & (16384 - 1)).reshape(ER, 128)
    dstp = (packed >> 14).reshape(ER, 128)
    spc = SUP * 128
    meta = jnp.stack([
        jnp.full((16,), c0 // spc, i32),
        jnp.full((16,), (c0 + spc - 1) // spc, i32),
    ])
    m1 = jnp.concatenate(
        [jnp.ones((N,), i32), jnp.zeros((NPAD - N,), i32)])
    z1 = jnp.zeros((NPAD,), f32)
    z2 = jnp.zeros((NHT, F), f32)

    H1, m2, x1r = _stage(xp, W1, b1, m1, srcp, dstp, meta, z1, z2, 5000)
    H2, m3, x2r = _stage(H1, W2, b2, m2, srcp, dstp, meta, z1, z2, 2500)
    H3, _, _ = _stage(H2, W3, b3, m3, srcp, dstp, meta, z1, z2, None)

    return _head(H3, m3.reshape(NPAD, 1), x1r, x2r,
                 L1W, L1b.reshape(1, F), L2W, L2b.reshape(1, F),
                 L3W, L3b.reshape(1, 64))


# speedup vs baseline: 1.3885x; 1.3885x over previous
import functools

import jax
import jax.numpy as jnp
from jax import lax
from jax.experimental import pallas as pl
from jax.experimental.pallas import tpu as pltpu
from jax.experimental.pallas import tpu_sc as plsc

N = 10000
NPAD = 10240
F = 128
E = 320000
EPAD = 327680
ER = EPAD // 128
NT = 32
TROWS = ER // NT
CHROWS = 4
NCH = TROWS // CHROWS
NSLICE = NPAD // 16
NH = NPAD // 2
NHT = NH + 128
RSLICE = NHT // 16
TROWS2 = ER // 16
NCH2 = TROWS2 // CHROWS


def _sc_deg_body(m_hbm, srcp, dstp, z1_hbm, degp_hbm, dste_hbm,
                 m_v, src_c, dst_c, d0_c, d1_c, val_c, sdeg):
    c = lax.axis_index("c")
    s = lax.axis_index("s")
    wid = c * 16 + s
    pltpu.sync_copy(m_hbm, m_v)
    pltpu.sync_copy(z1_hbm.at[pl.ds(s * NSLICE, NSLICE)],
                    sdeg.at[pl.ds(s * NSLICE, NSLICE)])
    plsc.subcore_barrier()

    def chunk(j, carry):
        rb = wid * TROWS + j * CHROWS
        pltpu.sync_copy(srcp.at[pl.ds(rb, CHROWS)], src_c)
        pltpu.sync_copy(dstp.at[pl.ds(rb, CHROWS)], dst_c)
        for g in range(CHROWS):
            for i in range(8):
                sl = pl.ds(i * 16, 16)
                s16 = src_c[g, sl]
                d16 = dst_c[g, sl]
                ms = plsc.load_gather(m_v, [s16])
                md = plsc.load_gather(m_v, [d16])
                vld = ms * md
                dump = NH + (d16 & 127)
                ok = vld > 0
                d0_c[g, sl] = jnp.where(ok & (d16 < NH), d16, dump)
                d1_c[g, sl] = jnp.where(ok & (d16 >= NH), d16 - NH, dump)
                val_c[g, sl] = vld.astype(jnp.float32)
        pltpu.sync_copy(d0_c, dste_hbm.at[0, pl.ds(rb, CHROWS)])
        pltpu.sync_copy(d1_c, dste_hbm.at[1, pl.ds(rb, CHROWS)])
        for g in range(CHROWS):
            pltpu.sync_copy(val_c.at[g], sdeg.at[dst_c.at[g]], add=True)
        return carry

    lax.fori_loop(0, NCH, chunk, 0)
    plsc.subcore_barrier()
    pltpu.sync_copy(sdeg.at[pl.ds(s * NSLICE, NSLICE)],
                    degp_hbm.at[c, pl.ds(s * NSLICE, NSLICE)])


@functools.lru_cache(maxsize=None)
def _get_sc_deg():
    mesh = plsc.VectorSubcoreMesh(core_axis_name="c", subcore_axis_name="s")
    return pl.kernel(
        _sc_deg_body,
        out_type=(
            jax.ShapeDtypeStruct((2, NPAD), jnp.float32),
            jax.ShapeDtypeStruct((2, ER, 128), jnp.int32),
        ),
        mesh=mesh,
        compiler_params=pltpu.CompilerParams(needs_layout_passes=False),
        scratch_types=[
            pltpu.VMEM((NPAD,), jnp.int32),
            pltpu.VMEM((CHROWS, 128), jnp.int32),
            pltpu.VMEM((CHROWS, 128), jnp.int32),
            pltpu.VMEM((CHROWS, 128), jnp.int32),
            pltpu.VMEM((CHROWS, 128), jnp.int32),
            pltpu.VMEM((CHROWS, 128), jnp.float32),
            pltpu.VMEM_SHARED((NPAD,), jnp.float32),
        ],
    )


SUP = 16
NSUPT = ER // SUP


def _sc_feat_body(tbl_hbm, srcp, dste, meta_hbm, z2_hbm, acc_hbm,
                  meta_s, src_c, dst_c, rows_a, rows_b, acc, gsem, ssem):
    c = lax.axis_index("c")
    s = lax.axis_index("s")
    pltpu.sync_copy(meta_hbm, meta_s)
    pltpu.sync_copy(z2_hbm.at[pl.ds(s * RSLICE, RSLICE)],
                    acc.at[pl.ds(s * RSLICE, RSLICE)])
    plsc.subcore_barrier()
    bufs = (rows_a, rows_b)
    t0f = jnp.max(meta_s[0, :])
    t0c = jnp.max(meta_s[1, :])
    start = jnp.where(c == 0, 0, t0f)
    end = jnp.where(c == 0, t0c, NSUPT)
    n = (end - start - s + 15) // 16

    def sup(j, carry):
        rb = (start + s + j * 16) * SUP
        pltpu.sync_copy(srcp.at[pl.ds(rb, SUP)], src_c)
        pltpu.sync_copy(dste.at[c, pl.ds(rb, SUP)], dst_c)
        pend_sc = [None, None]
        pend_g = [None, None]
        pend_g[0] = pltpu.async_copy(tbl_hbm.at[src_c.at[0]], bufs[0], gsem)
        for g in range(SUP):
            b = g % 2
            if g + 1 < SUP:
                nb = (g + 1) % 2
                if pend_sc[nb] is not None:
                    pend_sc[nb].wait()
                    pend_sc[nb] = None
                pend_g[nb] = pltpu.async_copy(
                    tbl_hbm.at[src_c.at[g + 1]], bufs[nb], gsem)
            pend_g[b].wait()
            pend_sc[b] = pltpu.async_copy(
                bufs[b], acc.at[dst_c.at[g]], ssem, add=True)
        for d in pend_sc:
            if d is not None:
                d.wait()
        return carry

    lax.fori_loop(0, n, sup, 0)
    plsc.subcore_barrier()
    pltpu.sync_copy(acc.at[pl.ds(s * RSLICE, RSLICE)],
                    acc_hbm.at[c, pl.ds(s * RSLICE, RSLICE)])


@functools.lru_cache(maxsize=None)
def _get_sc_feat():
    mesh = plsc.VectorSubcoreMesh(core_axis_name="c", subcore_axis_name="s")
    return pl.kernel(
        _sc_feat_body,
        out_type=jax.ShapeDtypeStruct((2, NHT, F), jnp.float32),
        mesh=mesh,
        compiler_params=pltpu.CompilerParams(needs_layout_passes=False),
        scratch_types=[
            pltpu.VMEM((2, 16), jnp.int32),
            pltpu.VMEM((SUP, 128), jnp.int32),
            pltpu.VMEM((SUP, 128), jnp.int32),
            pltpu.VMEM((128, F), jnp.float32),
            pltpu.VMEM((128, F), jnp.float32),
            pltpu.VMEM_SHARED((NHT, F), jnp.float32),
            pltpu.SemaphoreType.DMA,
            pltpu.SemaphoreType.DMA,
        ],
    )


EPADT = EPAD + 2048
PSL = EPADT // 16
TROWS3 = ER // 16
NCH3 = TROWS3 // CHROWS


def _sc_part_body(pk_hbm, pos_hbm, dum_hbm, meta_hbm, out_hbm,
                  meta_s, pk_c, pos_c, posm_c, buf):
    c = lax.axis_index("c")
    s = lax.axis_index("s")
    pltpu.sync_copy(meta_hbm, meta_s)
    pltpu.sync_copy(dum_hbm.at[pl.ds(s * PSL, PSL)],
                    buf.at[pl.ds(s * PSL, PSL)])
    plsc.subcore_barrier()
    r0c = jnp.max(meta_s[0, :])
    c0r = jnp.max(meta_s[1, :])
    cb = c != 0

    def chunk(j, carry):
        rb = s * TROWS3 + j * CHROWS
        pltpu.sync_copy(pk_hbm.at[pl.ds(rb, CHROWS)], pk_c)
        pltpu.sync_copy(pos_hbm.at[pl.ds(rb, CHROWS)], pos_c)
        for g in range(CHROWS):
            for i in range(8):
                sl = pl.ds(i * 16, 16)
                p16 = pos_c[g, sl]
                flag = jnp.logical_xor(p16 < c0r, cb)
                posm_c[g, sl] = jnp.where(flag, p16, EPAD)
        for g in range(CHROWS):
            pltpu.sync_copy(pk_c.at[g], buf.at[posm_c.at[g]])
        return carry

    lax.fori_loop(0, NCH3, chunk, 0)
    plsc.subcore_barrier()
    start = jnp.where(cb, r0c, 0)
    end = jnp.where(cb, ER, r0c)
    n = (end - start - s + 15) // 16

    def rower(j, carry):
        r = (start + s + j * 16) * 128
        pltpu.sync_copy(buf.at[pl.ds(r, 128)], out_hbm.at[pl.ds(r, 128)])
        return carry

    lax.fori_loop(0, n, rower, 0)


@functools.lru_cache(maxsize=None)
def _get_sc_part():
    mesh = plsc.VectorSubcoreMesh(core_axis_name="c", subcore_axis_name="s")
    return pl.kernel(
        _sc_part_body,
        out_type=jax.ShapeDtypeStruct((EPAD,), jnp.int32),
        mesh=mesh,
        compiler_params=pltpu.CompilerParams(needs_layout_passes=False),
        scratch_types=[
            pltpu.VMEM((2, 16), jnp.int32),
            pltpu.VMEM((CHROWS, 128), jnp.int32),
            pltpu.VMEM((CHROWS, 128), jnp.int32),
            pltpu.VMEM((CHROWS, 128), jnp.int32),
            pltpu.VMEM_SHARED((EPADT,), jnp.int32),
        ],
    )


BR = 512
GB = NPAD // BR
HB = NH // BR


def _dense_a_body(x_ref, w_ref, degp_ref, h_ref, ht_ref, degc_ref):
    deg = degp_ref[0] + degp_ref[1]
    h = jnp.dot(x_ref[...], w_ref[...], preferred_element_type=jnp.float32)
    dinvg = lax.rsqrt(deg + 1.0)
    h_ref[...] = h
    ht_ref[...] = h * dinvg
    degc_ref[...] = deg


_dense_a = pl.pallas_call(
    _dense_a_body,
    grid=(GB,),
    in_specs=[
        pl.BlockSpec((BR, F), lambda i: (i, 0)),
        pl.BlockSpec((F, F), lambda i: (0, 0)),
        pl.BlockSpec((2, BR, 1), lambda i: (0, i, 0)),
    ],
    out_specs=[
        pl.BlockSpec((BR, F), lambda i: (i, 0)),
        pl.BlockSpec((BR, F), lambda i: (i, 0)),
        pl.BlockSpec((BR, 1), lambda i: (i, 0)),
    ],
    out_shape=[
        jax.ShapeDtypeStruct((NPAD, F), jnp.float32),
        jax.ShapeDtypeStruct((NPAD, F), jnp.float32),
        jax.ShapeDtypeStruct((NPAD, 1), jnp.float32),
    ],
)


def _dense_b_body(h_ref, ap_ref, degc_ref, b_ref, H_ref, Hs_ref):
    deg = degc_ref[...]
    dinvg = lax.rsqrt(deg + 1.0)
    A = ap_ref[0]
    H = jnp.maximum(A * dinvg + h_ref[...] * (dinvg * dinvg) + b_ref[...], 0.0)
    H_ref[...] = H
    dinvs = jnp.where(deg > 0, lax.rsqrt(jnp.where(deg > 0, deg, 1.0)), 0.0)
    Hs_ref[...] = H * dinvs


_dense_b = pl.pallas_call(
    _dense_b_body,
    grid=(GB,),
    in_specs=[
        pl.BlockSpec((BR, F), lambda i: (i, 0)),
        pl.BlockSpec((1, BR, F), lambda i: (i // HB, i % HB, 0)),
        pl.BlockSpec((BR, 1), lambda i: (i, 0)),
        pl.BlockSpec((1, F), lambda i: (0, 0)),
    ],
    out_specs=[
        pl.BlockSpec((BR, F), lambda i: (i, 0)),
        pl.BlockSpec((BR, F), lambda i: (i, 0)),
    ],
    out_shape=[
        jax.ShapeDtypeStruct((NPAD, F), jnp.float32),
        jax.ShapeDtypeStruct((NPAD, F), jnp.float32),
    ],
)


def _pool_body(k, H_ref, pp_ref, degc_ref, m_ref, mn_ref, ro_ref):
    deg = degc_ref[...]
    dinvs = jnp.where(deg > 0, lax.rsqrt(jnp.where(deg > 0, deg, 1.0)), 0.0)
    H = H_ref[...]
    P = jnp.concatenate([pp_ref[0, :NH, :], pp_ref[1, :NH, :]], axis=0) * dinvs
    sc = jnp.sum(jnp.abs(H - P), axis=1, keepdims=True)
    key = jnp.where(m_ref[...] > 0, sc, -jnp.inf)
    bi = lax.bitcast_convert_type(key, jnp.int32)
    v = jnp.where(bi >= 0, bi, bi ^ jnp.int32(0x7FFFFFFF))
    w = lax.bitcast_convert_type(v ^ jnp.int32(-2147483648), jnp.uint32)

    def bit_body(b, acc):
        cand = acc | (jnp.uint32(1) << (jnp.uint32(31) - b.astype(jnp.uint32)))
        cnt = jnp.sum((w >= cand).astype(jnp.int32))
        return jnp.where(cnt >= k, cand, acc)

    ustar = lax.fori_loop(0, 32, bit_body, jnp.uint32(0))
    c_gt = jnp.sum((w > ustar).astype(jnp.int32))
    r = k - c_gt
    eq = w == ustar
    idx = lax.broadcasted_iota(jnp.int32, (NPAD, 1), 0)

    def ix_body(b, lohi):
        lo, hi = lohi
        mid = (lo + hi) // 2
        g = jnp.sum((eq & (idx < mid)).astype(jnp.int32))
        take = g >= r
        return (jnp.where(take, lo, mid), jnp.where(take, mid, hi))

    _, jstar = lax.fori_loop(0, 15, ix_body, (jnp.int32(0), jnp.int32(NPAD)))
    mn = (w > ustar) | (eq & (idx < jstar))
    mn_ref[...] = mn.astype(jnp.int32)
    mx = jnp.max(jnp.where(mn, H, -jnp.inf), axis=0, keepdims=True)
    sm = jnp.sum(jnp.where(mn, H, 0.0), axis=0, keepdims=True)
    ro_ref[...] = jnp.concatenate([mx, sm * (1.0 / k)], axis=1)


@functools.lru_cache(maxsize=None)
def _pool_call(k):
    return pl.pallas_call(
        functools.partial(_pool_body, k),
        out_shape=[
            jax.ShapeDtypeStruct((NPAD, 1), jnp.int32),
            jax.ShapeDtypeStruct((1, 2 * F), jnp.float32),
        ],
    )


def _head_body(H_ref, m_ref, x1_ref, x2_ref, w1_ref, v1_ref, w2_ref, v2_ref,
               w3_ref, v3_ref, out_ref):
    mb = m_ref[...] > 0
    H = H_ref[...]
    mx = jnp.max(jnp.where(mb, H, -jnp.inf), axis=0, keepdims=True)
    sm = jnp.sum(jnp.where(mb, H, 0.0), axis=0, keepdims=True)
    x3 = jnp.concatenate([mx, sm * (1.0 / 2500.0)], axis=1)
    z = (jnp.maximum(x1_ref[...], 0.0) + jnp.maximum(x2_ref[...], 0.0)
         + jnp.maximum(x3, 0.0))
    z = jnp.maximum(
        jnp.dot(z, w1_ref[...], preferred_element_type=jnp.float32)
        + v1_ref[...], 0.0)
    z = jnp.maximum(
        jnp.dot(z, w2_ref[...], preferred_element_type=jnp.float32)
        + v2_ref[...], 0.0)
    z = (jnp.dot(z, w3_ref[...], preferred_element_type=jnp.float32)
         + v3_ref[...])
    nrm = jnp.maximum(jnp.sqrt(jnp.sum(z * z)), 1e-12)
    out_ref[...] = z / nrm


_head = pl.pallas_call(
    _head_body,
    out_shape=jax.ShapeDtypeStruct((1, 64), jnp.float32),
)


def _stage(X, W, bvec, m_flat, srcp, dstp, meta, z1, z2, k):
    degp, dste = _get_sc_deg()(m_flat, srcp, dstp, z1)
    h, ht, degc = _dense_a(X, W, degp.reshape(2, NPAD, 1))
    Ap = _get_sc_feat()(ht, srcp, dste, meta, z2)
    H, Hs = _dense_b(h, Ap, degc, bvec.reshape(1, F))
    if k is None:
        return H, None, None
    Pp = _get_sc_feat()(Hs, srcp, dste, meta, z2)
    mn, ro = _pool_call(k)(H, Pp, degc, m_flat.reshape(NPAD, 1))
    return H, mn.reshape(NPAD), ro


def kernel(x, edge_index, batch, W1, b1, W2, b2, W3, b3,
           L1W, L1b, L2W, L2b, L3W, L3b):
    f32 = jnp.float32
    i32 = jnp.int32
    xp = jnp.zeros((NPAD, F), f32).at[:N].set(x)
    src = edge_index[0].astype(i32)
    dst = edge_index[1].astype(i32)
    in0 = dst < NH
    f0 = in0.astype(i32)
    f2 = f0.reshape(E // 128, 128)
    ws = jnp.sum(f2, axis=1)
    cw = jnp.cumsum(ws)
    lane = jnp.cumsum(f2, axis=1)
    cs0 = (lane + (cw - ws)[:, None]).reshape(E)
    c0 = cw[E // 128 - 1]
    ii = jnp.arange(E, dtype=i32)
    c0r = ((c0 + 127) // 128) * 128
    pos = jnp.where(in0, cs0 - 1, c0r + ii - cs0)
    dummy = (NPAD - 1) << 14
    pk0 = jnp.concatenate(
        [(dst << 14) | src,
         jnp.full((EPAD - E,), dummy, i32)]).reshape(ER, 128)
    posq = jnp.concatenate(
        [pos, jnp.full((EPAD - E,), EPAD, i32)]).reshape(ER, 128)
    pmeta = jnp.stack([jnp.full((16,), c0r // 128, i32),
                       jnp.full((16,), c0r, i32)])
    dum = jnp.full((EPADT,), dummy, i32)
    packed = _get_sc_part()(pk0, posq, dum, pmeta)
    srcp = (packed & (16384 - 1)).reshape(ER, 128)
    dstp = (packed >> 14).reshape(ER, 128)
    spc = SUP * 128
    meta = jnp.stack([
        jnp.full((16,), c0 // spc, i32),
        jnp.full((16,), (c0 + spc - 1) // spc, i32),
    ])
    m1 = jnp.concatenate(
        [jnp.ones((N,), i32), jnp.zeros((NPAD - N,), i32)])
    z1 = jnp.zeros((NPAD,), f32)
    z2 = jnp.zeros((NHT, F), f32)

    H1, m2, x1r = _stage(xp, W1, b1, m1, srcp, dstp, meta, z1, z2, 5000)
    H2, m3, x2r = _stage(H1, W2, b2, m2, srcp, dstp, meta, z1, z2, 2500)
    H3, _, _ = _stage(H2, W3, b3, m3, srcp, dstp, meta, z1, z2, None)

    return _head(H3, m3.reshape(NPAD, 1), x1r, x2r,
                 L1W, L1b.reshape(1, F), L2W, L2b.reshape(1, F),
                 L3W, L3b.reshape(1, 64))
